# Initial kernel scaffold; baseline (speedup 1.0000x reference)
import functools

import jax
import jax.numpy as jnp
from jax import lax
from jax.experimental import pallas as pl
from jax.experimental.pallas import tpu as pltpu
from jax.experimental.pallas import tpu_sc as plsc

NUM_CORES = 2
NUM_SUBCORES = 16
LANES = 16
NW = NUM_CORES * NUM_SUBCORES

PAIRS = 768
ROWS = PAIRS // NW
DIM = 12
PERIODS = 288
XHEAD = 16


def _full(v):
    return jnp.full((LANES,), v, jnp.int32)


def _sc_body(x2, te, ptab, wtab, out, xbuf, tebuf, ptabv, wtabv, outbuf, sem):
    wid = lax.axis_index("s") * NUM_CORES + lax.axis_index("c")
    base = wid * ROWS

    c1 = pltpu.async_copy(x2.at[pl.ds(base, ROWS), pl.ds(0, XHEAD)], xbuf, sem)
    c2 = pltpu.async_copy(te.at[pl.ds(base, ROWS), :], tebuf, sem)
    c3 = pltpu.async_copy(ptab, ptabv, sem)
    c4 = pltpu.async_copy(wtab, wtabv, sem)
    c1.wait()
    c2.wait()
    c3.wait()
    c4.wait()

    iota = lax.iota(jnp.int32, LANES)
    for blk in range(2):
        r = iota + blk * LANES
        valid = r < ROWS
        rc = jnp.minimum(r, ROWS - 1)
        xp = plsc.load_gather(xbuf, [rc, _full(1)])
        xw = plsc.load_gather(xbuf, [rc, _full(2)])
        pidx = jnp.clip((xp * float(PERIODS)).astype(jnp.int32), 0, PERIODS - 1)
        widx = jnp.clip(xw.astype(jnp.int32), 0, 6)
        for d in range(DIM):
            dv = _full(d)
            pe = plsc.load_gather(ptabv, [pidx, dv])
            we = plsc.load_gather(wtabv, [widx, dv])
            tev = plsc.load_gather(tebuf, [rc, dv])
            res = tev * pe * we
            if blk == 0:
                plsc.store_scatter(outbuf, [rc, dv], res)
            else:
                plsc.store_scatter(outbuf, [rc, dv], res, mask=valid)

    pltpu.sync_copy(outbuf, out.at[pl.ds(base, ROWS), :])


_sc_encoder = functools.partial(
    pl.kernel,
    mesh=plsc.VectorSubcoreMesh(core_axis_name="c", subcore_axis_name="s"),
    out_type=jax.ShapeDtypeStruct((PAIRS, DIM), jnp.float32),
    scratch_types=[
        pltpu.VMEM((ROWS, XHEAD), jnp.float32),
        pltpu.VMEM((ROWS, DIM), jnp.float32),
        pltpu.VMEM((PERIODS, DIM), jnp.float32),
        pltpu.VMEM((8, DIM), jnp.float32),
        pltpu.VMEM((ROWS, DIM), jnp.float32),
        pltpu.SemaphoreType.DMA,
    ],
)(_sc_body)


def kernel(x, periods_table, weekend_table, node_embeddings, time_embeddings):
    b, t, n, c = x.shape
    x2 = x.reshape(b * t, n * c)
    te = time_embeddings[:b].reshape(b * t, DIM)
    wtab = jnp.pad(weekend_table, ((0, 1), (0, 0)))
    out = _sc_encoder(x2, te, periods_table, wtab)
    return node_embeddings, out.reshape(b, t, DIM)


# speedup vs baseline: 1562.8686x; 1562.8686x over previous
import functools

import jax
import jax.numpy as jnp
from jax import lax
from jax.experimental import pallas as pl
from jax.experimental.pallas import tpu as pltpu
from jax.experimental.pallas import tpu_sc as plsc

NUM_CORES = 2
NUM_SUBCORES = 16
LANES = 16
NW = NUM_CORES * NUM_SUBCORES

PAIRS = 768
ROWS = PAIRS // NW
DIM = 12
PERIODS = 288


def _floor_idx(v, hi):
    i = v.astype(jnp.int32)
    i = jnp.where(i.astype(jnp.float32) > v, i - 1, i)
    return jnp.clip(i, 0, hi)


def _sc_body(xs, xw, te, ptab, wtab, out, xsv, xwv, tebuf, ptabv, wtabv,
             outbuf, sem):
    wid = lax.axis_index("s") * NUM_CORES + lax.axis_index("c")
    base = wid * ROWS

    c1 = pltpu.async_copy(xs.at[pl.ds(base, ROWS)], xsv.at[pl.ds(0, ROWS)], sem)
    c2 = pltpu.async_copy(xw.at[pl.ds(base, ROWS)], xwv.at[pl.ds(0, ROWS)], sem)
    c3 = pltpu.async_copy(te.at[pl.ds(base, ROWS), :], tebuf, sem)
    c4 = pltpu.async_copy(ptab, ptabv, sem)
    c5 = pltpu.async_copy(wtab, wtabv, sem)
    c1.wait()
    c2.wait()
    c3.wait()
    c4.wait()
    c5.wait()

    pidx = []
    widx = []
    for blk in range(2):
        pv = xsv[pl.ds(blk * LANES, LANES)]
        wv = xwv[pl.ds(blk * LANES, LANES)]
        pidx.append(_floor_idx(pv * float(PERIODS), PERIODS - 1))
        widx.append(_floor_idx(wv, 6))

    for r in range(ROWS):
        blk, lane = divmod(r, LANES)
        pe = ptabv[pidx[blk][lane], :]
        we = wtabv[widx[blk][lane], :]
        outbuf[r, :] = tebuf[r, :] * pe * we

    pltpu.sync_copy(outbuf, out.at[pl.ds(base, ROWS), :])


_sc_encoder = functools.partial(
    pl.kernel,
    mesh=plsc.VectorSubcoreMesh(core_axis_name="c", subcore_axis_name="s"),
    out_type=jax.ShapeDtypeStruct((PAIRS, LANES), jnp.float32),
    scratch_types=[
        pltpu.VMEM((2 * LANES,), jnp.float32),
        pltpu.VMEM((2 * LANES,), jnp.float32),
        pltpu.VMEM((ROWS, LANES), jnp.float32),
        pltpu.VMEM((PERIODS, LANES), jnp.float32),
        pltpu.VMEM((8, LANES), jnp.float32),
        pltpu.VMEM((ROWS, LANES), jnp.float32),
        pltpu.SemaphoreType.DMA,
    ],
)(_sc_body)


def kernel(x, periods_table, weekend_table, node_embeddings, time_embeddings):
    b, t = x.shape[0], x.shape[1]
    xs = x[:, :, 0, 1].reshape(b * t)
    xw = x[:, :, 0, 2].reshape(b * t)
    te = jnp.pad(time_embeddings[:b].reshape(b * t, DIM), ((0, 0), (0, LANES - DIM)))
    ptab = jnp.pad(periods_table, ((0, 0), (0, LANES - DIM)))
    wtab = jnp.pad(weekend_table, ((0, 1), (0, LANES - DIM)))
    out = _sc_encoder(xs, xw, te, ptab, wtab)
    return node_embeddings, out[:, :DIM].reshape(b, t, DIM)


# speedup vs baseline: 1601.8175x; 1.0249x over previous
import functools

import jax
import jax.numpy as jnp
from jax import lax
from jax.experimental import pallas as pl
from jax.experimental.pallas import tpu as pltpu
from jax.experimental.pallas import tpu_sc as plsc

NUM_CORES = 2
NUM_SUBCORES = 16
LANES = 16
NW = NUM_CORES * NUM_SUBCORES

PAIRS = 768
ROWS = PAIRS // NW
DIM = 12
PERIODS = 288
TROWS = PERIODS + 8


def _floor_idx(v, lo_rows, hi):
    i = v.astype(jnp.int32)
    i = jnp.where(i.astype(jnp.float32) > v, i - 1, i)
    return jnp.clip(i, 0, hi) + lo_rows


def _sc_body(ax, tab, out, axv, tabv, outbuf, sem):
    wid = lax.axis_index("s") * NUM_CORES + lax.axis_index("c")
    base = wid * ROWS

    c1 = pltpu.async_copy(ax.at[pl.ds(base, ROWS), :], axv, sem)
    c2 = pltpu.async_copy(tab, tabv, sem)
    c1.wait()
    c2.wait()

    for r in range(ROWS):
        row = axv[r, :]
        pidx = _floor_idx(row[0] * float(PERIODS), 0, PERIODS - 1)
        widx = _floor_idx(row[1], PERIODS, 6)
        outbuf[r, :] = row * tabv[pidx, :] * tabv[widx, :]

    pltpu.sync_copy(outbuf, out.at[pl.ds(base, ROWS), :])


_sc_encoder = functools.partial(
    pl.kernel,
    mesh=plsc.VectorSubcoreMesh(core_axis_name="c", subcore_axis_name="s"),
    out_type=jax.ShapeDtypeStruct((PAIRS, LANES), jnp.float32),
    scratch_types=[
        pltpu.VMEM((ROWS, LANES), jnp.float32),
        pltpu.VMEM((TROWS, LANES), jnp.float32),
        pltpu.VMEM((ROWS, LANES), jnp.float32),
        pltpu.SemaphoreType.DMA,
    ],
)(_sc_body)


def kernel(x, periods_table, weekend_table, node_embeddings, time_embeddings):
    b, t = x.shape[0], x.shape[1]
    n = b * t
    ax = jnp.concatenate(
        [
            x[:, :, 0, 1:3].reshape(n, 2),
            time_embeddings[:b].reshape(n, DIM),
            jnp.zeros((n, LANES - DIM - 2), jnp.float32),
        ],
        axis=1,
    )
    tab = jnp.pad(
        jnp.concatenate([periods_table, jnp.pad(weekend_table, ((0, 1), (0, 0)))]),
        ((0, 0), (2, LANES - DIM - 2)),
    )
    out = _sc_encoder(ax, tab)
    return node_embeddings, out[:, 2 : 2 + DIM].reshape(b, t, DIM)


# speedup vs baseline: 1868.2650x; 1.1663x over previous
import functools

import jax
import jax.numpy as jnp
from jax import lax
from jax.experimental import pallas as pl
from jax.experimental.pallas import tpu as pltpu
from jax.experimental.pallas import tpu_sc as plsc

NUM_CORES = 2
NUM_SUBCORES = 16
LANES = 16
NW = NUM_CORES * NUM_SUBCORES

PAIRS = 768
ROWS = PAIRS // NW
DIM = 12
PERIODS = 288
PT_WORDS = PERIODS * DIM
WT_WORDS = 7 * DIM


def _floor_off(v, hi):
    i = v.astype(jnp.int32)
    i = jnp.where(i.astype(jnp.float32) > v, i - 1, i)
    return jnp.clip(i, 0, hi) * DIM


def _sc_body(xsw, tef, ptf, wtf, out, xbuf, pibuf, wibuf, tebuf, ptbuf,
             wtbuf, outbuf, sem):
    wid = lax.axis_index("s") * NUM_CORES + lax.axis_index("c")
    base = wid * ROWS

    c1 = pltpu.async_copy(xsw.at[pl.ds(base, ROWS)], xbuf.at[pl.ds(0, ROWS)], sem)
    c2 = pltpu.async_copy(xsw.at[pl.ds(PAIRS + base, ROWS)],
                          xbuf.at[pl.ds(32, ROWS)], sem)
    c3 = pltpu.async_copy(tef.at[pl.ds(base * DIM, ROWS * DIM)],
                          tebuf.at[pl.ds(0, ROWS * DIM)], sem)
    c4 = pltpu.async_copy(ptf, ptbuf.at[pl.ds(0, PT_WORDS)], sem)
    c5 = pltpu.async_copy(wtf, wtbuf.at[pl.ds(0, WT_WORDS)], sem)
    c1.wait()
    c2.wait()

    for off in (0, ROWS - LANES):
        pv = xbuf[pl.ds(off, LANES)]
        wv = xbuf[pl.ds(32 + off, LANES)]
        pibuf[pl.ds(off, LANES)] = _floor_off(pv * float(PERIODS), PERIODS - 1)
        wibuf[pl.ds(off, LANES)] = _floor_off(wv, 6)

    c3.wait()
    c4.wait()
    c5.wait()

    def body(r, carry):
        p12 = pibuf[pl.ds(r, LANES)][0]
        w12 = wibuf[pl.ds(r, LANES)][0]
        tev = tebuf[pl.ds(r * DIM, LANES)]
        pe = ptbuf[pl.ds(p12, LANES)]
        we = wtbuf[pl.ds(w12, LANES)]
        outbuf[pl.ds(r * DIM, LANES)] = tev * pe * we
        return carry

    lax.fori_loop(0, ROWS, body, 0)

    pltpu.sync_copy(outbuf.at[pl.ds(0, ROWS * DIM)],
                    out.at[pl.ds(base * DIM, ROWS * DIM)])


_sc_encoder = functools.partial(
    pl.kernel,
    mesh=plsc.VectorSubcoreMesh(core_axis_name="c", subcore_axis_name="s"),
    out_type=jax.ShapeDtypeStruct((PAIRS * DIM,), jnp.float32),
    scratch_types=[
        pltpu.VMEM((64,), jnp.float32),
        pltpu.VMEM((ROWS + LANES,), jnp.int32),
        pltpu.VMEM((ROWS + LANES,), jnp.int32),
        pltpu.VMEM((ROWS * DIM + LANES,), jnp.float32),
        pltpu.VMEM((PT_WORDS + LANES,), jnp.float32),
        pltpu.VMEM((WT_WORDS + LANES,), jnp.float32),
        pltpu.VMEM((ROWS * DIM + LANES,), jnp.float32),
        pltpu.SemaphoreType.DMA,
    ],
)(_sc_body)


def kernel(x, periods_table, weekend_table, node_embeddings, time_embeddings):
    b, t = x.shape[0], x.shape[1]
    xsw = x[:, :, 0, 1:3].transpose(2, 0, 1).reshape(2 * b * t)
    tef = time_embeddings[:b].reshape(b * t * DIM)
    ptf = periods_table.reshape(PT_WORDS)
    wtf = weekend_table.reshape(WT_WORDS)
    out = _sc_encoder(xsw, tef, ptf, wtf)
    return node_embeddings, out.reshape(b, t, DIM)
